# baseline (speedup 1.0000x reference)
import functools

import jax
import jax.numpy as jnp
from jax import lax
from jax.experimental import pallas as pl
from jax.experimental.pallas import tpu as pltpu
from jax.experimental.pallas import tpu_sc as plsc

_N = 10000
_D = 128
_E = 320000
_NC = 2
_NS = 16
_NW = _NC * _NS
_L = 16

_NP = 10240
_NTILE = _NP // _NS
_EP = 331776
_EPT = _EP // _NW
_RPT = _E // _NW
_SPT = _NP // _NW
_AC = 128
_ACPT = _EPT // _AC
_EC = 64
_CPT = _EPT // _EC
_SCR = 18
_SCE = _SCR * _EC
_NPAIR = _SCR // 2

_mesh = plsc.VectorSubcoreMesh(core_axis_name="c", subcore_axis_name="s")
_params = pltpu.CompilerParams(needs_layout_passes=False, use_tc_tiling_on_sc=False)


def _rsqrt_vec(d):
    i = lax.bitcast_convert_type(d, jnp.int32)
    y = lax.bitcast_convert_type(jnp.int32(0x5F3759DF) - (i >> 1), jnp.float32)
    half_d = 0.5 * d
    y = y * (1.5 - half_d * y * y)
    y = y * (1.5 - half_d * y * y)
    return y


def _zeros16():
    return jnp.zeros((_L,), jnp.float32)


@functools.partial(
    pl.kernel,
    out_type=(
        jax.ShapeDtypeStruct((_NC * _NP,), jnp.float32),
        jax.ShapeDtypeStruct((_EP,), jnp.int32),
        jax.ShapeDtypeStruct((_EP,), jnp.int32),
        jax.ShapeDtypeStruct((_EP,), jnp.float32),
    ),
    mesh=_mesh,
    scratch_types=[
        pltpu.VMEM_SHARED((_NP,), jnp.float32),
        pltpu.VMEM((_EPT,), jnp.int32),
        pltpu.VMEM((_EPT,), jnp.int32),
        pltpu.VMEM((_EPT,), jnp.float32),
        pltpu.VMEM((_AC,), jnp.int32),
        pltpu.VMEM((_AC,), jnp.float32),
        pltpu.VMEM((_NTILE,), jnp.float32),
    ],
    compiler_params=_params,
)
def _deg_kernel(ei_hbm, w_hbm, deg_out, rows_out, cols_out, weff_out,
                deg_sh, rstage, cstage, wstage, ridx, weff, zbuf):
    cid = lax.axis_index("c")
    sid = lax.axis_index("s")
    wid = cid * _NS + sid

    def _zero(i, carry):
        zbuf[pl.ds(i * _L, _L)] = _zeros16()
        return carry

    lax.fori_loop(0, _NTILE // _L, _zero, 0)
    nsl = pl.ds(sid * _NTILE, _NTILE)
    pltpu.sync_copy(zbuf, deg_sh.at[nsl])

    pltpu.sync_copy(ei_hbm.at[pl.ds(wid * _RPT, _RPT)], rstage.at[pl.ds(0, _RPT)])
    pltpu.sync_copy(ei_hbm.at[pl.ds(_E + wid * _RPT, _RPT)], cstage.at[pl.ds(0, _RPT)])
    pltpu.sync_copy(w_hbm.at[pl.ds(wid * _RPT, _RPT)], wstage.at[pl.ds(0, _RPT)])
    plsc.subcore_barrier()

    iota = lax.iota(jnp.int32, _L)
    selfbase = wid * _SPT - _RPT

    def _chunk(j, carry):
        for q in range(_AC // _L):
            toff = j * _AC + q * _L
            sl = pl.ds(toff, _L)
            is_real = toff < _RPT
            is_self = toff < _RPT + _SPT
            r16 = rstage[sl]
            c16 = cstage[sl]
            w16 = wstage[sl]
            wr = jnp.where(r16 != c16, w16, _zeros16())
            idraw = selfbase + toff + iota
            id_eff = jnp.where(idraw < _N, idraw, idraw - _N)
            w_syn = jnp.where((idraw < _N) & is_self,
                              jnp.full((_L,), 1.0, jnp.float32), _zeros16())
            rows16 = jnp.where(is_real, r16, id_eff)
            cols16 = jnp.where(is_real, c16, id_eff)
            weff16 = jnp.where(is_real, wr, w_syn)
            rstage[sl] = rows16
            cstage[sl] = cols16
            wstage[sl] = weff16
            qsl = pl.ds(q * _L, _L)
            ridx[qsl] = rows16
            weff[qsl] = weff16
        pltpu.sync_copy(weff, deg_sh.at[ridx], add=True)
        return carry

    lax.fori_loop(0, _ACPT, _chunk, 0)

    esl = pl.ds(wid * _EPT, _EPT)
    pltpu.sync_copy(rstage, rows_out.at[esl])
    pltpu.sync_copy(cstage, cols_out.at[esl])
    pltpu.sync_copy(wstage, weff_out.at[esl])

    plsc.subcore_barrier()
    pltpu.sync_copy(deg_sh.at[nsl], deg_out.at[pl.ds(cid * _NP + sid * _NTILE, _NTILE)])


@functools.partial(
    pl.kernel,
    out_type=jax.ShapeDtypeStruct((_NC * _NP, _D), jnp.float32),
    mesh=_mesh,
    scratch_types=[
        pltpu.VMEM_SHARED((_NP, _D), jnp.float32),
        pltpu.VMEM_SHARED((_NP,), jnp.float32),
        pltpu.VMEM((_NP,), jnp.float32),
        pltpu.VMEM((_SCE,), jnp.int32),
        pltpu.VMEM((_SCE,), jnp.int32),
        pltpu.VMEM((_SCE,), jnp.float32),
        pltpu.VMEM((2, _SCR, _EC), jnp.int32),
        pltpu.VMEM((_EC,), jnp.int32),
        pltpu.VMEM((_EC, _D), jnp.bfloat16),
        pltpu.VMEM((_EC, _D), jnp.bfloat16),
        pltpu.VMEM((_EC, _D), jnp.float32),
        pltpu.VMEM((_EC, _D), jnp.float32),
        pltpu.VMEM((_NTILE,), jnp.float32),
        pltpu.VMEM((_NTILE,), jnp.float32),
        pltpu.SemaphoreType.DMA,
        pltpu.SemaphoreType.DMA,
        pltpu.SemaphoreType.DMA,
        pltpu.SemaphoreType.DMA,
    ],
    compiler_params=_params,
)
def _spmm_kernel(deg_hbm, row_hbm, col_hbm, w_hbm, xb_hbm, out_hbm,
                 out_sh, dis_sh, dis_v, rstage, cstage, wstage,
                 ridx3, pidx, bbuf0, bbuf1, sbuf0, sbuf1, dbuf0, dbuf1,
                 semg0, semg1, sems0, sems1):
    cid = lax.axis_index("c")
    sid = lax.axis_index("s")
    wid = cid * _NS + sid

    izero = jnp.zeros((_L,), jnp.int32)

    def _zero(j, carry):
        for q in range(_D // _L):
            sl = pl.ds(q * _L, _L)
            sbuf0[j, sl] = _zeros16()
            sbuf1[j, sl] = _zeros16()
        return carry

    lax.fori_loop(0, _EC, _zero, 0)
    for q in range(_EC // _L):
        pidx[pl.ds(q * _L, _L)] = izero

    for k in range(_NTILE // _EC):
        pltpu.sync_copy(sbuf0, out_sh.at[pl.ds(sid * _NTILE + k * _EC, _EC)])

    nsl = pl.ds(sid * _NTILE, _NTILE)
    pltpu.sync_copy(deg_hbm.at[pl.ds(sid * _NTILE, _NTILE)], dbuf0)
    pltpu.sync_copy(deg_hbm.at[pl.ds(_NP + sid * _NTILE, _NTILE)], dbuf1)

    def _dis(i, carry):
        sl = pl.ds(i * _L, _L)
        d = dbuf0[sl] + dbuf1[sl]
        d = jnp.where(d > 0.0, d, d + 1.0)
        dbuf0[sl] = _rsqrt_vec(d)
        return carry

    lax.fori_loop(0, _NTILE // _L, _dis, 0)
    pltpu.sync_copy(dbuf0, dis_sh.at[nsl])
    plsc.subcore_barrier()
    pltpu.sync_copy(dis_sh, dis_v)

    pltpu.async_copy(sbuf0, out_sh.at[pidx], sems0, add=True)
    pltpu.async_copy(sbuf1, out_sh.at[pidx], sems1, add=True)

    def _wait_g(sem, dst):
        pltpu.make_async_copy(xb_hbm.at[pl.ds(0, _EC)], dst, sem).wait()

    def _wait_s(sem, dst):
        pltpu.make_async_copy(out_hbm.at[pl.ds(0, _EC)], dst, sem).wait()

    def _scale(bb, sb, j):
        def _g(g, carry):
            sl = pl.ds(j * _EC + g * _L, _L)
            r16 = rstage[sl]
            c16 = cstage[sl]
            w16 = wstage[sl]
            wn16 = plsc.load_gather(dis_v, [r16]) * w16 * plsc.load_gather(dis_v, [c16])
            for e16 in range(_L):
                s = wn16[e16]
                e = g * _L + e16
                for t in range(_D // (2 * _L)):
                    v32 = bb[e, pl.ds(t * 2 * _L, 2 * _L)]
                    va, vb = plsc.unpack(v32, format=plsc.PackFormat.INTERLEAVED)
                    sb[e, pl.ds(t * 2 * _L, _L)] = va * s
                    sb[e, pl.ds(t * 2 * _L + _L, _L)] = vb * s
            return carry

        lax.fori_loop(0, _EC // _L, _g, 0)

    base = wid * _EPT
    for sc in range(_CPT // _SCR):
        par = sc % 2
        sbase = base + sc * _SCE
        pltpu.sync_copy(row_hbm.at[pl.ds(sbase, _SCE)], rstage)
        pltpu.sync_copy(col_hbm.at[pl.ds(sbase, _SCE)], cstage)
        pltpu.sync_copy(w_hbm.at[pl.ds(sbase, _SCE)], wstage)

        def _bld(j, carry):
            for q in range(_EC // _L):
                ridx3[par, j, pl.ds(q * _L, _L)] = rstage[pl.ds(j * _EC + q * _L, _L)]
            return carry

        lax.fori_loop(0, _SCR, _bld, 0)

        pltpu.async_copy(xb_hbm.at[cstage.at[pl.ds(0, _EC)]], bbuf0, semg0)

        def _pair(p, carry):
            a = 2 * p
            b = a + 1
            pltpu.async_copy(
                xb_hbm.at[cstage.at[pl.ds(b * _EC, _EC)]], bbuf1, semg1)
            _wait_s(sems0, sbuf0)
            _wait_g(semg0, bbuf0)
            _scale(bbuf0, sbuf0, a)
            pltpu.async_copy(sbuf0, out_sh.at[ridx3.at[par, a]], sems0, add=True)

            @pl.when(p < _NPAIR - 1)
            def _prefetch():
                pltpu.async_copy(
                    xb_hbm.at[cstage.at[pl.ds((a + 2) * _EC, _EC)]], bbuf0, semg0)

            _wait_s(sems1, sbuf1)
            _wait_g(semg1, bbuf1)
            _scale(bbuf1, sbuf1, b)
            pltpu.async_copy(sbuf1, out_sh.at[ridx3.at[par, b]], sems1, add=True)
            return carry

        lax.fori_loop(0, _NPAIR, _pair, 0)

    _wait_s(sems1, sbuf1)
    _wait_s(sems0, sbuf0)
    plsc.subcore_barrier()
    pltpu.sync_copy(out_sh.at[nsl], out_hbm.at[pl.ds(cid * _NP + sid * _NTILE, _NTILE)])


def _add_body(a_ref, b_ref, o_ref):
    o_ref[...] = a_ref[...] + b_ref[...]


_BR = 1000


def _combine(p0, p1):
    return pl.pallas_call(
        _add_body,
        out_shape=jax.ShapeDtypeStruct((_N, _D), jnp.float32),
        grid=(_N // _BR,),
        in_specs=[pl.BlockSpec((_BR, _D), lambda i: (i, 0))] * 2,
        out_specs=pl.BlockSpec((_BR, _D), lambda i: (i, 0)),
    )(p0, p1)


def kernel(x, edge_index, edge_attr):
    assert x.shape == (_N, _D)
    ei = edge_index.astype(jnp.int32).reshape(2 * _E)
    w = edge_attr.astype(jnp.float32)
    xb = (x.reshape(_N, _D // 32, 2, _L)
          .transpose(0, 1, 3, 2)
          .reshape(_N, _D)
          .astype(jnp.bfloat16))

    deg, rows, cols, weff = _deg_kernel(ei, w)
    parts = _spmm_kernel(deg, rows, cols, weff, xb)
    return _combine(parts[:_N], parts[_NP:_NP + _N])


# speedup vs baseline: 1.4116x; 1.4116x over previous
import functools

import jax
import jax.numpy as jnp
from jax import lax
from jax.experimental import pallas as pl
from jax.experimental.pallas import tpu as pltpu
from jax.experimental.pallas import tpu_sc as plsc

_N = 10000
_D = 128
_E = 320000
_NC = 2
_NS = 16
_NW = _NC * _NS
_L = 16

_NP = 10240
_NTILE = _NP // _NS
_EP = 344064
_EPT = _EP // _NW
_RPT = _E // _NW
_SPT = _NP // _NW
_AC = 128
_ACPT = _EPT // _AC
_EC = 128
_CPT = _EPT // _EC
_SCR = 6
_SCE = _SCR * _EC
_NPAIR = _SCR // 2

_mesh = plsc.VectorSubcoreMesh(core_axis_name="c", subcore_axis_name="s")
_params = pltpu.CompilerParams(needs_layout_passes=False, use_tc_tiling_on_sc=False)


def _rsqrt_vec(d):
    i = lax.bitcast_convert_type(d, jnp.int32)
    y = lax.bitcast_convert_type(jnp.int32(0x5F3759DF) - (i >> 1), jnp.float32)
    half_d = 0.5 * d
    y = y * (1.5 - half_d * y * y)
    y = y * (1.5 - half_d * y * y)
    return y


def _zeros16():
    return jnp.zeros((_L,), jnp.float32)


@functools.partial(
    pl.kernel,
    out_type=(
        jax.ShapeDtypeStruct((_NC * _NP,), jnp.float32),
        jax.ShapeDtypeStruct((_EP,), jnp.int32),
        jax.ShapeDtypeStruct((_EP,), jnp.int32),
        jax.ShapeDtypeStruct((_EP,), jnp.float32),
    ),
    mesh=_mesh,
    scratch_types=[
        pltpu.VMEM_SHARED((_NP,), jnp.float32),
        pltpu.VMEM((_EPT,), jnp.int32),
        pltpu.VMEM((_EPT,), jnp.int32),
        pltpu.VMEM((_EPT,), jnp.float32),
        pltpu.VMEM((_AC,), jnp.int32),
        pltpu.VMEM((_AC,), jnp.float32),
        pltpu.VMEM((_NTILE,), jnp.float32),
    ],
    compiler_params=_params,
)
def _deg_kernel(ei_hbm, w_hbm, deg_out, rows_out, cols_out, weff_out,
                deg_sh, rstage, cstage, wstage, ridx, weff, zbuf):
    cid = lax.axis_index("c")
    sid = lax.axis_index("s")
    wid = cid * _NS + sid

    def _zero(i, carry):
        zbuf[pl.ds(i * _L, _L)] = _zeros16()
        return carry

    lax.fori_loop(0, _NTILE // _L, _zero, 0)
    nsl = pl.ds(sid * _NTILE, _NTILE)
    pltpu.sync_copy(zbuf, deg_sh.at[nsl])

    pltpu.sync_copy(ei_hbm.at[pl.ds(wid * _RPT, _RPT)], rstage.at[pl.ds(0, _RPT)])
    pltpu.sync_copy(ei_hbm.at[pl.ds(_E + wid * _RPT, _RPT)], cstage.at[pl.ds(0, _RPT)])
    pltpu.sync_copy(w_hbm.at[pl.ds(wid * _RPT, _RPT)], wstage.at[pl.ds(0, _RPT)])
    plsc.subcore_barrier()

    iota = lax.iota(jnp.int32, _L)
    selfbase = wid * _SPT - _RPT

    def _chunk(j, carry):
        for q in range(_AC // _L):
            toff = j * _AC + q * _L
            sl = pl.ds(toff, _L)
            is_real = toff < _RPT
            is_self = toff < _RPT + _SPT
            r16 = rstage[sl]
            c16 = cstage[sl]
            w16 = wstage[sl]
            wr = jnp.where(r16 != c16, w16, _zeros16())
            idraw = selfbase + toff + iota
            id_eff = jnp.where(idraw < _N, idraw, idraw - _N)
            w_syn = jnp.where((idraw < _N) & is_self,
                              jnp.full((_L,), 1.0, jnp.float32), _zeros16())
            rows16 = jnp.where(is_real, r16, id_eff)
            cols16 = jnp.where(is_real, c16, id_eff)
            weff16 = jnp.where(is_real, wr, w_syn)
            rstage[sl] = rows16
            cstage[sl] = cols16
            wstage[sl] = weff16
            qsl = pl.ds(q * _L, _L)
            ridx[qsl] = rows16
            weff[qsl] = weff16
        pltpu.sync_copy(weff, deg_sh.at[ridx], add=True)
        return carry

    lax.fori_loop(0, _ACPT, _chunk, 0)

    esl = pl.ds(wid * _EPT, _EPT)
    pltpu.sync_copy(rstage, rows_out.at[esl])
    pltpu.sync_copy(cstage, cols_out.at[esl])
    pltpu.sync_copy(wstage, weff_out.at[esl])

    plsc.subcore_barrier()
    pltpu.sync_copy(deg_sh.at[nsl], deg_out.at[pl.ds(cid * _NP + sid * _NTILE, _NTILE)])


@functools.partial(
    pl.kernel,
    out_type=jax.ShapeDtypeStruct((_NC * _NP, _D), jnp.float32),
    mesh=_mesh,
    scratch_types=[
        pltpu.VMEM_SHARED((_NP, _D), jnp.float32),
        pltpu.VMEM_SHARED((_NP,), jnp.float32),
        pltpu.VMEM((_NP,), jnp.float32),
        pltpu.VMEM((_SCE,), jnp.int32),
        pltpu.VMEM((_SCE,), jnp.int32),
        pltpu.VMEM((_SCE,), jnp.float32),
        pltpu.VMEM((2, _SCR, _EC), jnp.int32),
        pltpu.VMEM((_EC,), jnp.int32),
        pltpu.VMEM((_EC, _D), jnp.float32),
        pltpu.VMEM((_EC, _D), jnp.float32),
        pltpu.VMEM((_NTILE,), jnp.float32),
        pltpu.VMEM((_NTILE,), jnp.float32),
        pltpu.SemaphoreType.DMA,
        pltpu.SemaphoreType.DMA,
        pltpu.SemaphoreType.DMA,
        pltpu.SemaphoreType.DMA,
    ],
    compiler_params=_params,
)
def _spmm_kernel(deg_hbm, row_hbm, col_hbm, w_hbm, x_hbm, out_hbm,
                 out_sh, dis_sh, dis_v, rstage, cstage, wstage,
                 ridx3, pidx, buf0, buf1, dbuf0, dbuf1,
                 semg0, semg1, sems0, sems1):
    cid = lax.axis_index("c")
    sid = lax.axis_index("s")
    wid = cid * _NS + sid

    izero = jnp.zeros((_L,), jnp.int32)

    def _zero(j, carry):
        for q in range(_D // _L):
            sl = pl.ds(q * _L, _L)
            buf0[j, sl] = _zeros16()
            buf1[j, sl] = _zeros16()
        return carry

    lax.fori_loop(0, _EC, _zero, 0)
    for q in range(_EC // _L):
        pidx[pl.ds(q * _L, _L)] = izero

    for k in range(_NTILE // _EC):
        pltpu.sync_copy(buf0, out_sh.at[pl.ds(sid * _NTILE + k * _EC, _EC)])

    nsl = pl.ds(sid * _NTILE, _NTILE)
    pltpu.sync_copy(deg_hbm.at[pl.ds(sid * _NTILE, _NTILE)], dbuf0)
    pltpu.sync_copy(deg_hbm.at[pl.ds(_NP + sid * _NTILE, _NTILE)], dbuf1)

    def _dis(i, carry):
        sl = pl.ds(i * _L, _L)
        d = dbuf0[sl] + dbuf1[sl]
        d = jnp.where(d > 0.0, d, d + 1.0)
        dbuf0[sl] = _rsqrt_vec(d)
        return carry

    lax.fori_loop(0, _NTILE // _L, _dis, 0)
    pltpu.sync_copy(dbuf0, dis_sh.at[nsl])
    plsc.subcore_barrier()
    pltpu.sync_copy(dis_sh, dis_v)

    pltpu.async_copy(buf1, out_sh.at[pidx], sems1, add=True)

    def _wait(sem, dst):
        pltpu.make_async_copy(x_hbm.at[pl.ds(0, _EC)], dst, sem).wait()

    def _scale(buf, j):
        def _g(g, carry):
            sl = pl.ds(j * _EC + g * _L, _L)
            r16 = rstage[sl]
            c16 = cstage[sl]
            w16 = wstage[sl]
            wn16 = plsc.load_gather(dis_v, [r16]) * w16 * plsc.load_gather(dis_v, [c16])
            for e16 in range(_L):
                s = wn16[e16]
                e = g * _L + e16
                for q in range(_D // _L):
                    qsl = pl.ds(q * _L, _L)
                    buf[e, qsl] = buf[e, qsl] * s
            return carry

        lax.fori_loop(0, _EC // _L, _g, 0)

    base = wid * _EPT
    for sc in range(_CPT // _SCR):
        par = sc % 2
        sbase = base + sc * _SCE
        pltpu.sync_copy(row_hbm.at[pl.ds(sbase, _SCE)], rstage)
        pltpu.sync_copy(col_hbm.at[pl.ds(sbase, _SCE)], cstage)
        pltpu.sync_copy(w_hbm.at[pl.ds(sbase, _SCE)], wstage)

        def _bld(j, carry):
            for q in range(_EC // _L):
                ridx3[par, j, pl.ds(q * _L, _L)] = rstage[pl.ds(j * _EC + q * _L, _L)]
            return carry

        lax.fori_loop(0, _SCR, _bld, 0)

        pltpu.async_copy(x_hbm.at[cstage.at[pl.ds(0, _EC)]], buf0, semg0)

        def _pair(p, carry):
            a = 2 * p
            b = a + 1
            _wait(sems1, buf1)
            pltpu.async_copy(
                x_hbm.at[cstage.at[pl.ds(b * _EC, _EC)]], buf1, semg1)
            _wait(semg0, buf0)
            _scale(buf0, a)
            pltpu.async_copy(buf0, out_sh.at[ridx3.at[par, a]], sems0, add=True)
            _wait(semg1, buf1)
            _scale(buf1, b)
            _wait(sems0, buf0)

            @pl.when(p < _NPAIR - 1)
            def _prefetch():
                pltpu.async_copy(
                    x_hbm.at[cstage.at[pl.ds((a + 2) * _EC, _EC)]], buf0, semg0)

            pltpu.async_copy(buf1, out_sh.at[ridx3.at[par, b]], sems1, add=True)
            return carry

        lax.fori_loop(0, _NPAIR, _pair, 0)

    _wait(sems1, buf1)
    plsc.subcore_barrier()
    pltpu.sync_copy(out_sh.at[nsl], out_hbm.at[pl.ds(cid * _NP + sid * _NTILE, _NTILE)])


def _add_body(a_ref, b_ref, o_ref):
    o_ref[...] = a_ref[...] + b_ref[...]


_BR = 1000


def _combine(p0, p1):
    return pl.pallas_call(
        _add_body,
        out_shape=jax.ShapeDtypeStruct((_N, _D), jnp.float32),
        grid=(_N // _BR,),
        in_specs=[pl.BlockSpec((_BR, _D), lambda i: (i, 0))] * 2,
        out_specs=pl.BlockSpec((_BR, _D), lambda i: (i, 0)),
    )(p0, p1)


def kernel(x, edge_index, edge_attr):
    assert x.shape == (_N, _D)
    ei = edge_index.astype(jnp.int32).reshape(2 * _E)
    w = edge_attr.astype(jnp.float32)

    deg, rows, cols, weff = _deg_kernel(ei, w)
    parts = _spmm_kernel(deg, rows, cols, weff, x)
    return _combine(parts[:_N], parts[_NP:_NP + _N])


# speedup vs baseline: 1.5814x; 1.1203x over previous
import functools

import jax
import jax.numpy as jnp
from jax import lax
from jax.experimental import pallas as pl
from jax.experimental.pallas import tpu as pltpu
from jax.experimental.pallas import tpu_sc as plsc

_N = 10000
_D = 128
_E = 320000
_NC = 2
_NS = 16
_NW = _NC * _NS
_L = 16

_NP = 10240
_NTILE = _NP // _NS
_EP = 331776
_EPT = _EP // _NW
_RPT = _E // _NW
_SPT = _NP // _NW
_AC = 128
_ACPT = _EPT // _AC
_EC = 96
_CPT = _EPT // _EC
_SCR = 18
_SCE = _SCR * _EC
_NPAIR = _SCR // 2

_mesh = plsc.VectorSubcoreMesh(core_axis_name="c", subcore_axis_name="s")
_params = pltpu.CompilerParams(needs_layout_passes=False, use_tc_tiling_on_sc=False)


def _rsqrt_vec(d):
    i = lax.bitcast_convert_type(d, jnp.int32)
    y = lax.bitcast_convert_type(jnp.int32(0x5F3759DF) - (i >> 1), jnp.float32)
    half_d = 0.5 * d
    y = y * (1.5 - half_d * y * y)
    y = y * (1.5 - half_d * y * y)
    return y


def _zeros16():
    return jnp.zeros((_L,), jnp.float32)


@functools.partial(
    pl.kernel,
    out_type=(
        jax.ShapeDtypeStruct((_NC * _NP,), jnp.float32),
        jax.ShapeDtypeStruct((_EP,), jnp.int32),
        jax.ShapeDtypeStruct((_EP,), jnp.int32),
        jax.ShapeDtypeStruct((_EP,), jnp.float32),
    ),
    mesh=_mesh,
    scratch_types=[
        pltpu.VMEM_SHARED((_NP,), jnp.float32),
        pltpu.VMEM((_EPT,), jnp.int32),
        pltpu.VMEM((_EPT,), jnp.int32),
        pltpu.VMEM((_EPT,), jnp.float32),
        pltpu.VMEM((_AC,), jnp.int32),
        pltpu.VMEM((_AC,), jnp.float32),
        pltpu.VMEM((_NTILE,), jnp.float32),
    ],
    compiler_params=_params,
)
def _deg_kernel(ei_hbm, w_hbm, deg_out, rows_out, cols_out, weff_out,
                deg_sh, rstage, cstage, wstage, ridx, weff, zbuf):
    cid = lax.axis_index("c")
    sid = lax.axis_index("s")
    wid = cid * _NS + sid

    def _zero(i, carry):
        zbuf[pl.ds(i * _L, _L)] = _zeros16()
        return carry

    lax.fori_loop(0, _NTILE // _L, _zero, 0)
    nsl = pl.ds(sid * _NTILE, _NTILE)
    pltpu.sync_copy(zbuf, deg_sh.at[nsl])

    pltpu.sync_copy(ei_hbm.at[pl.ds(wid * _RPT, _RPT)], rstage.at[pl.ds(0, _RPT)])
    pltpu.sync_copy(ei_hbm.at[pl.ds(_E + wid * _RPT, _RPT)], cstage.at[pl.ds(0, _RPT)])
    pltpu.sync_copy(w_hbm.at[pl.ds(wid * _RPT, _RPT)], wstage.at[pl.ds(0, _RPT)])
    plsc.subcore_barrier()

    iota = lax.iota(jnp.int32, _L)
    selfbase = wid * _SPT - _RPT

    def _chunk(j, carry):
        for q in range(_AC // _L):
            toff = j * _AC + q * _L
            sl = pl.ds(toff, _L)
            is_real = toff < _RPT
            is_self = toff < _RPT + _SPT
            r16 = rstage[sl]
            c16 = cstage[sl]
            w16 = wstage[sl]
            wr = jnp.where(r16 != c16, w16, _zeros16())
            idraw = selfbase + toff + iota
            id_eff = jnp.where(idraw < _N, idraw, idraw - _N)
            w_syn = jnp.where((idraw < _N) & is_self,
                              jnp.full((_L,), 1.0, jnp.float32), _zeros16())
            rows16 = jnp.where(is_real, r16, id_eff)
            cols16 = jnp.where(is_real, c16, id_eff)
            weff16 = jnp.where(is_real, wr, w_syn)
            rstage[sl] = rows16
            cstage[sl] = cols16
            wstage[sl] = weff16
            qsl = pl.ds(q * _L, _L)
            ridx[qsl] = rows16
            weff[qsl] = weff16
        pltpu.sync_copy(weff, deg_sh.at[ridx], add=True)
        return carry

    lax.fori_loop(0, _ACPT, _chunk, 0)

    esl = pl.ds(wid * _EPT, _EPT)
    pltpu.sync_copy(rstage, rows_out.at[esl])
    pltpu.sync_copy(cstage, cols_out.at[esl])
    pltpu.sync_copy(wstage, weff_out.at[esl])

    plsc.subcore_barrier()
    pltpu.sync_copy(deg_sh.at[nsl], deg_out.at[pl.ds(cid * _NP + sid * _NTILE, _NTILE)])


@functools.partial(
    pl.kernel,
    out_type=jax.ShapeDtypeStruct((_NC * _NP, _D), jnp.float32),
    mesh=_mesh,
    scratch_types=[
        pltpu.VMEM_SHARED((_NP, _D), jnp.float32),
        pltpu.VMEM_SHARED((_NP,), jnp.float32),
        pltpu.VMEM((_NP,), jnp.float32),
        pltpu.VMEM((_SCE,), jnp.int32),
        pltpu.VMEM((_SCE,), jnp.int32),
        pltpu.VMEM((_SCE,), jnp.float32),
        pltpu.VMEM((2, _SCR, _EC), jnp.int32),
        pltpu.VMEM((_EC,), jnp.int32),
        pltpu.VMEM((_EC, _D), jnp.float32),
        pltpu.VMEM((_EC, _D), jnp.float32),
        pltpu.VMEM((_NTILE,), jnp.float32),
        pltpu.VMEM((_NTILE,), jnp.float32),
        pltpu.SemaphoreType.DMA,
        pltpu.SemaphoreType.DMA,
        pltpu.SemaphoreType.DMA,
        pltpu.SemaphoreType.DMA,
    ],
    compiler_params=_params,
)
def _spmm_kernel(deg_hbm, row_hbm, col_hbm, w_hbm, x_hbm, out_hbm,
                 out_sh, dis_sh, dis_v, rstage, cstage, wstage,
                 ridx3, pidx, buf0, buf1, dbuf0, dbuf1,
                 semg0, semg1, sems0, sems1):
    cid = lax.axis_index("c")
    sid = lax.axis_index("s")
    wid = cid * _NS + sid

    izero = jnp.zeros((_L,), jnp.int32)

    def _zero(j, carry):
        for q in range(_D // _L):
            sl = pl.ds(q * _L, _L)
            buf0[j, sl] = _zeros16()
            buf1[j, sl] = _zeros16()
        return carry

    lax.fori_loop(0, _EC, _zero, 0)
    for q in range(_EC // _L):
        pidx[pl.ds(q * _L, _L)] = izero

    for k in range(6):
        pltpu.sync_copy(buf0, out_sh.at[pl.ds(sid * _NTILE + k * _EC, _EC)])
    pltpu.sync_copy(buf0.at[pl.ds(0, _NTILE - 6 * _EC)],
                    out_sh.at[pl.ds(sid * _NTILE + 6 * _EC, _NTILE - 6 * _EC)])

    nsl = pl.ds(sid * _NTILE, _NTILE)
    pltpu.sync_copy(deg_hbm.at[pl.ds(sid * _NTILE, _NTILE)], dbuf0)
    pltpu.sync_copy(deg_hbm.at[pl.ds(_NP + sid * _NTILE, _NTILE)], dbuf1)

    def _dis(i, carry):
        sl = pl.ds(i * _L, _L)
        d = dbuf0[sl] + dbuf1[sl]
        d = jnp.where(d > 0.0, d, d + 1.0)
        dbuf0[sl] = _rsqrt_vec(d)
        return carry

    lax.fori_loop(0, _NTILE // _L, _dis, 0)
    pltpu.sync_copy(dbuf0, dis_sh.at[nsl])
    plsc.subcore_barrier()
    pltpu.sync_copy(dis_sh, dis_v)

    pltpu.async_copy(buf1, out_sh.at[pidx], sems1, add=True)

    def _wait(sem, dst):
        pltpu.make_async_copy(x_hbm.at[pl.ds(0, _EC)], dst, sem).wait()

    def _scale(buf, j):
        def _g(g, carry):
            sl = pl.ds(j * _EC + g * _L, _L)
            r16 = rstage[sl]
            c16 = cstage[sl]
            w16 = wstage[sl]
            wn16 = plsc.load_gather(dis_v, [r16]) * w16 * plsc.load_gather(dis_v, [c16])
            for e16 in range(_L):
                s = wn16[e16]
                e = g * _L + e16
                for q in range(_D // _L):
                    qsl = pl.ds(q * _L, _L)
                    buf[e, qsl] = buf[e, qsl] * s
            return carry

        lax.fori_loop(0, _EC // _L, _g, 0)

    base = wid * _EPT
    for sc in range(_CPT // _SCR):
        par = sc % 2
        sbase = base + sc * _SCE
        pltpu.sync_copy(row_hbm.at[pl.ds(sbase, _SCE)], rstage)
        pltpu.sync_copy(col_hbm.at[pl.ds(sbase, _SCE)], cstage)
        pltpu.sync_copy(w_hbm.at[pl.ds(sbase, _SCE)], wstage)

        def _bld(j, carry):
            for q in range(_EC // _L):
                ridx3[par, j, pl.ds(q * _L, _L)] = rstage[pl.ds(j * _EC + q * _L, _L)]
            return carry

        lax.fori_loop(0, _SCR, _bld, 0)

        pltpu.async_copy(x_hbm.at[cstage.at[pl.ds(0, _EC)]], buf0, semg0)

        def _pair(p, carry):
            a = 2 * p
            b = a + 1
            _wait(sems1, buf1)
            pltpu.async_copy(
                x_hbm.at[cstage.at[pl.ds(b * _EC, _EC)]], buf1, semg1)
            _wait(semg0, buf0)
            _scale(buf0, a)
            pltpu.async_copy(buf0, out_sh.at[ridx3.at[par, a]], sems0, add=True)
            _wait(semg1, buf1)
            _scale(buf1, b)
            _wait(sems0, buf0)

            @pl.when(p < _NPAIR - 1)
            def _prefetch():
                pltpu.async_copy(
                    x_hbm.at[cstage.at[pl.ds((a + 2) * _EC, _EC)]], buf0, semg0)

            pltpu.async_copy(buf1, out_sh.at[ridx3.at[par, b]], sems1, add=True)
            return carry

        lax.fori_loop(0, _NPAIR, _pair, 0)

    _wait(sems1, buf1)
    plsc.subcore_barrier()
    pltpu.sync_copy(out_sh.at[nsl], out_hbm.at[pl.ds(cid * _NP + sid * _NTILE, _NTILE)])


def _add_body(a_ref, b_ref, o_ref):
    o_ref[...] = a_ref[...] + b_ref[...]


_BR = 1000


def _combine(p0, p1):
    return pl.pallas_call(
        _add_body,
        out_shape=jax.ShapeDtypeStruct((_N, _D), jnp.float32),
        grid=(_N // _BR,),
        in_specs=[pl.BlockSpec((_BR, _D), lambda i: (i, 0))] * 2,
        out_specs=pl.BlockSpec((_BR, _D), lambda i: (i, 0)),
    )(p0, p1)


def kernel(x, edge_index, edge_attr):
    assert x.shape == (_N, _D)
    ei = edge_index.astype(jnp.int32).reshape(2 * _E)
    w = edge_attr.astype(jnp.float32)

    deg, rows, cols, weff = _deg_kernel(ei, w)
    parts = _spmm_kernel(deg, rows, cols, weff, x)
    return _combine(parts[:_N], parts[_NP:_NP + _N])


# speedup vs baseline: 1.6083x; 1.0170x over previous
import functools

import jax
import jax.numpy as jnp
from jax import lax
from jax.experimental import pallas as pl
from jax.experimental.pallas import tpu as pltpu
from jax.experimental.pallas import tpu_sc as plsc

_N = 10000
_D = 128
_E = 320000
_NC = 2
_NS = 16
_NW = _NC * _NS
_L = 16

_NP = 10240
_NTILE = _NP // _NS
_EP = 331776
_EPT = _EP // _NW
_RPT = _E // _NW
_SPT = _NP // _NW
_AC = 128
_ACPT = _EPT // _AC
_EC = 96
_CPT = _EPT // _EC
_SCR = 18
_SCE = _SCR * _EC
_NPAIR = _SCR // 2

_mesh = plsc.VectorSubcoreMesh(core_axis_name="c", subcore_axis_name="s")
_params = pltpu.CompilerParams(needs_layout_passes=False, use_tc_tiling_on_sc=False)


def _rsqrt_vec(d):
    i = lax.bitcast_convert_type(d, jnp.int32)
    y = lax.bitcast_convert_type(jnp.int32(0x5F3759DF) - (i >> 1), jnp.float32)
    half_d = 0.5 * d
    y = y * (1.5 - half_d * y * y)
    y = y * (1.5 - half_d * y * y)
    return y


def _zeros16():
    return jnp.zeros((_L,), jnp.float32)


@functools.partial(
    pl.kernel,
    out_type=(
        jax.ShapeDtypeStruct((_NC * _NP,), jnp.float32),
        jax.ShapeDtypeStruct((_EP,), jnp.int32),
        jax.ShapeDtypeStruct((_EP,), jnp.int32),
        jax.ShapeDtypeStruct((_EP,), jnp.float32),
    ),
    mesh=_mesh,
    scratch_types=[
        pltpu.VMEM_SHARED((_NP,), jnp.float32),
        pltpu.VMEM((_EPT,), jnp.int32),
        pltpu.VMEM((_EPT,), jnp.int32),
        pltpu.VMEM((_EPT,), jnp.float32),
        pltpu.VMEM((_AC,), jnp.int32),
        pltpu.VMEM((_AC,), jnp.float32),
        pltpu.VMEM((_NTILE,), jnp.float32),
    ],
    compiler_params=_params,
)
def _deg_kernel(ei_hbm, w_hbm, deg_out, rows_out, cols_out, weff_out,
                deg_sh, rstage, cstage, wstage, ridx, weff, zbuf):
    cid = lax.axis_index("c")
    sid = lax.axis_index("s")
    wid = cid * _NS + sid

    def _zero(i, carry):
        zbuf[pl.ds(i * _L, _L)] = _zeros16()
        return carry

    lax.fori_loop(0, _NTILE // _L, _zero, 0)
    nsl = pl.ds(sid * _NTILE, _NTILE)
    pltpu.sync_copy(zbuf, deg_sh.at[nsl])

    pltpu.sync_copy(ei_hbm.at[pl.ds(wid * _RPT, _RPT)], rstage.at[pl.ds(0, _RPT)])
    pltpu.sync_copy(ei_hbm.at[pl.ds(_E + wid * _RPT, _RPT)], cstage.at[pl.ds(0, _RPT)])
    pltpu.sync_copy(w_hbm.at[pl.ds(wid * _RPT, _RPT)], wstage.at[pl.ds(0, _RPT)])
    plsc.subcore_barrier()

    iota = lax.iota(jnp.int32, _L)
    selfbase = wid * _SPT - _RPT

    def _chunk(j, carry):
        for q in range(_AC // _L):
            toff = j * _AC + q * _L
            sl = pl.ds(toff, _L)
            is_real = toff < _RPT
            is_self = toff < _RPT + _SPT
            r16 = rstage[sl]
            c16 = cstage[sl]
            w16 = wstage[sl]
            wr = jnp.where(r16 != c16, w16, _zeros16())
            idraw = selfbase + toff + iota
            id_eff = jnp.where(idraw < _N, idraw, idraw - _N)
            w_syn = jnp.where((idraw < _N) & is_self,
                              jnp.full((_L,), 1.0, jnp.float32), _zeros16())
            rows16 = jnp.where(is_real, r16, id_eff)
            cols16 = jnp.where(is_real, c16, id_eff)
            weff16 = jnp.where(is_real, wr, w_syn)
            rstage[sl] = rows16
            cstage[sl] = cols16
            wstage[sl] = weff16
            qsl = pl.ds(q * _L, _L)
            ridx[qsl] = rows16
            weff[qsl] = weff16
        pltpu.sync_copy(weff, deg_sh.at[ridx], add=True)
        return carry

    lax.fori_loop(0, _ACPT, _chunk, 0)

    esl = pl.ds(wid * _EPT, _EPT)
    pltpu.sync_copy(rstage, rows_out.at[esl])
    pltpu.sync_copy(cstage, cols_out.at[esl])
    pltpu.sync_copy(wstage, weff_out.at[esl])

    plsc.subcore_barrier()
    pltpu.sync_copy(deg_sh.at[nsl], deg_out.at[pl.ds(cid * _NP + sid * _NTILE, _NTILE)])


@functools.partial(
    pl.kernel,
    out_type=jax.ShapeDtypeStruct((_NC * _NP, _D), jnp.float32),
    mesh=_mesh,
    scratch_types=[
        pltpu.VMEM_SHARED((_NP, _D), jnp.float32),
        pltpu.VMEM_SHARED((_NP,), jnp.float32),
        pltpu.VMEM((_NP,), jnp.float32),
        pltpu.VMEM((_SCE,), jnp.int32),
        pltpu.VMEM((_SCE,), jnp.int32),
        pltpu.VMEM((_SCE,), jnp.float32),
        pltpu.VMEM((2, _SCR, _EC), jnp.int32),
        pltpu.VMEM((_EC,), jnp.int32),
        pltpu.VMEM((_EC, _D), jnp.float32),
        pltpu.VMEM((_EC, _D), jnp.float32),
        pltpu.VMEM((_NTILE,), jnp.float32),
        pltpu.VMEM((_NTILE,), jnp.float32),
        pltpu.SemaphoreType.DMA,
        pltpu.SemaphoreType.DMA,
        pltpu.SemaphoreType.DMA,
        pltpu.SemaphoreType.DMA,
    ],
    compiler_params=_params,
)
def _spmm_kernel(deg_hbm, row_hbm, col_hbm, w_hbm, x_hbm, out_hbm,
                 out_sh, dis_sh, dis_v, rstage, cstage, wstage,
                 ridx3, pidx, buf0, buf1, dbuf0, dbuf1,
                 semg0, semg1, sems0, sems1):
    cid = lax.axis_index("c")
    sid = lax.axis_index("s")
    wid = cid * _NS + sid

    izero = jnp.zeros((_L,), jnp.int32)

    def _zero(j, carry):
        for q in range(_D // _L):
            sl = pl.ds(q * _L, _L)
            buf0[j, sl] = _zeros16()
            buf1[j, sl] = _zeros16()
        return carry

    lax.fori_loop(0, _EC, _zero, 0)
    for q in range(_EC // _L):
        pidx[pl.ds(q * _L, _L)] = izero

    for k in range(6):
        pltpu.sync_copy(buf0, out_sh.at[pl.ds(sid * _NTILE + k * _EC, _EC)])
    pltpu.sync_copy(buf0.at[pl.ds(0, _NTILE - 6 * _EC)],
                    out_sh.at[pl.ds(sid * _NTILE + 6 * _EC, _NTILE - 6 * _EC)])

    nsl = pl.ds(sid * _NTILE, _NTILE)
    pltpu.sync_copy(deg_hbm.at[pl.ds(sid * _NTILE, _NTILE)], dbuf0)
    pltpu.sync_copy(deg_hbm.at[pl.ds(_NP + sid * _NTILE, _NTILE)], dbuf1)

    def _dis(i, carry):
        sl = pl.ds(i * _L, _L)
        d = dbuf0[sl] + dbuf1[sl]
        d = jnp.where(d > 0.0, d, d + 1.0)
        dbuf0[sl] = _rsqrt_vec(d)
        return carry

    lax.fori_loop(0, _NTILE // _L, _dis, 0)
    pltpu.sync_copy(dbuf0, dis_sh.at[nsl])
    plsc.subcore_barrier()
    pltpu.sync_copy(dis_sh, dis_v)

    pltpu.async_copy(buf1, out_sh.at[pidx], sems1, add=True)

    def _wait(sem, dst):
        pltpu.make_async_copy(x_hbm.at[pl.ds(0, _EC)], dst, sem).wait()

    def _scale(buf, j):
        def _g(g, carry):
            sl = pl.ds(j * _EC + g * _L, _L)
            r16 = rstage[sl]
            c16 = cstage[sl]
            w16 = wstage[sl]
            wn16 = plsc.load_gather(dis_v, [r16]) * w16 * plsc.load_gather(dis_v, [c16])
            for e16 in range(_L):
                s = wn16[e16]
                e = g * _L + e16
                for q in range(_D // _L):
                    qsl = pl.ds(q * _L, _L)
                    buf[e, qsl] = buf[e, qsl] * s
            return carry

        lax.fori_loop(0, _EC // _L, _g, 0)

    base = wid * _EPT
    for sc in range(_CPT // _SCR):
        par = sc % 2
        sbase = base + sc * _SCE
        pltpu.sync_copy(row_hbm.at[pl.ds(sbase, _SCE)], rstage)
        pltpu.sync_copy(col_hbm.at[pl.ds(sbase, _SCE)], cstage)
        pltpu.sync_copy(w_hbm.at[pl.ds(sbase, _SCE)], wstage)

        def _bld(j, carry):
            for q in range(_EC // _L):
                ridx3[par, j, pl.ds(q * _L, _L)] = rstage[pl.ds(j * _EC + q * _L, _L)]
            return carry

        lax.fori_loop(0, _SCR, _bld, 0)

        pltpu.async_copy(x_hbm.at[cstage.at[pl.ds(0, _EC)]], buf0, semg0)

        def _pair(p, carry):
            a = 2 * p
            b = a + 1
            _wait(sems1, buf1)
            pltpu.async_copy(
                x_hbm.at[cstage.at[pl.ds(b * _EC, _EC)]], buf1, semg1)
            _wait(semg0, buf0)
            _scale(buf0, a)
            pltpu.async_copy(buf0, out_sh.at[ridx3.at[par, a]], sems0, add=True)
            _wait(semg1, buf1)
            _wait(sems0, buf0)

            @pl.when(p < _NPAIR - 1)
            def _prefetch():
                pltpu.async_copy(
                    x_hbm.at[cstage.at[pl.ds((a + 2) * _EC, _EC)]], buf0, semg0)

            _scale(buf1, b)
            pltpu.async_copy(buf1, out_sh.at[ridx3.at[par, b]], sems1, add=True)
            return carry

        lax.fori_loop(0, _NPAIR, _pair, 0)

    _wait(sems1, buf1)
    plsc.subcore_barrier()
    pltpu.sync_copy(out_sh.at[nsl], out_hbm.at[pl.ds(cid * _NP + sid * _NTILE, _NTILE)])


def _add_body(a_ref, b_ref, o_ref):
    o_ref[...] = a_ref[...] + b_ref[...]


_BR = 1000


def _combine(p0, p1):
    return pl.pallas_call(
        _add_body,
        out_shape=jax.ShapeDtypeStruct((_N, _D), jnp.float32),
        grid=(_N // _BR,),
        in_specs=[pl.BlockSpec((_BR, _D), lambda i: (i, 0))] * 2,
        out_specs=pl.BlockSpec((_BR, _D), lambda i: (i, 0)),
    )(p0, p1)


def kernel(x, edge_index, edge_attr):
    assert x.shape == (_N, _D)
    ei = edge_index.astype(jnp.int32).reshape(2 * _E)
    w = edge_attr.astype(jnp.float32)

    deg, rows, cols, weff = _deg_kernel(ei, w)
    parts = _spmm_kernel(deg, rows, cols, weff, x)
    return _combine(parts[:_N], parts[_NP:_NP + _N])


# speedup vs baseline: 1.6198x; 1.0072x over previous
import functools

import jax
import jax.numpy as jnp
from jax import lax
from jax.experimental import pallas as pl
from jax.experimental.pallas import tpu as pltpu
from jax.experimental.pallas import tpu_sc as plsc

_N = 10000
_D = 128
_E = 320000
_NC = 2
_NS = 16
_NW = _NC * _NS
_L = 16

_NP = 10240
_NTILE = _NP // _NS
_EP = 331776
_EPT = _EP // _NW
_RPT = _E // _NW
_SPT = _NP // _NW
_AC = 128
_ACPT = _EPT // _AC
_EC = 96
_CPT = _EPT // _EC
_SCR = 18
_SCE = _SCR * _EC
_NPAIR = _SCR // 2

_mesh = plsc.VectorSubcoreMesh(core_axis_name="c", subcore_axis_name="s")
_params = pltpu.CompilerParams(needs_layout_passes=False, use_tc_tiling_on_sc=False)


def _rsqrt_vec(d):
    i = lax.bitcast_convert_type(d, jnp.int32)
    y = lax.bitcast_convert_type(jnp.int32(0x5F3759DF) - (i >> 1), jnp.float32)
    half_d = 0.5 * d
    y = y * (1.5 - half_d * y * y)
    y = y * (1.5 - half_d * y * y)
    return y


def _zeros16():
    return jnp.zeros((_L,), jnp.float32)


@functools.partial(
    pl.kernel,
    out_type=(
        jax.ShapeDtypeStruct((_NC * _NP,), jnp.float32),
        jax.ShapeDtypeStruct((_EP,), jnp.int32),
        jax.ShapeDtypeStruct((_EP,), jnp.int32),
        jax.ShapeDtypeStruct((_EP,), jnp.float32),
    ),
    mesh=_mesh,
    scratch_types=[
        pltpu.VMEM_SHARED((_NP,), jnp.float32),
        pltpu.VMEM((_EPT,), jnp.int32),
        pltpu.VMEM((_EPT,), jnp.int32),
        pltpu.VMEM((_EPT,), jnp.float32),
        pltpu.VMEM((_AC,), jnp.int32),
        pltpu.VMEM((_AC,), jnp.float32),
        pltpu.VMEM((_NTILE,), jnp.float32),
        pltpu.SemaphoreType.DMA,
    ],
    compiler_params=_params,
)
def _deg_kernel(ei_hbm, w_hbm, deg_out, rows_out, cols_out, weff_out,
                deg_sh, rstage, cstage, wstage, ridx, weff, zbuf, sem):
    cid = lax.axis_index("c")
    sid = lax.axis_index("s")
    wid = cid * _NS + sid

    def _zero(i, carry):
        zbuf[pl.ds(i * _L, _L)] = _zeros16()
        return carry

    lax.fori_loop(0, _NTILE // _L, _zero, 0)
    nsl = pl.ds(sid * _NTILE, _NTILE)
    pltpu.sync_copy(zbuf, deg_sh.at[nsl])

    d1 = pltpu.async_copy(ei_hbm.at[pl.ds(wid * _RPT, _RPT)],
                          rstage.at[pl.ds(0, _RPT)], sem)
    d2 = pltpu.async_copy(ei_hbm.at[pl.ds(_E + wid * _RPT, _RPT)],
                          cstage.at[pl.ds(0, _RPT)], sem)
    d3 = pltpu.async_copy(w_hbm.at[pl.ds(wid * _RPT, _RPT)],
                          wstage.at[pl.ds(0, _RPT)], sem)
    d1.wait()
    d2.wait()
    d3.wait()
    plsc.subcore_barrier()

    iota = lax.iota(jnp.int32, _L)
    selfbase = wid * _SPT - _RPT

    def _chunk(j, carry):
        for q in range(_AC // _L):
            toff = j * _AC + q * _L
            sl = pl.ds(toff, _L)
            is_real = toff < _RPT
            is_self = toff < _RPT + _SPT
            r16 = rstage[sl]
            c16 = cstage[sl]
            w16 = wstage[sl]
            wr = jnp.where(r16 != c16, w16, _zeros16())
            idraw = selfbase + toff + iota
            id_eff = jnp.where(idraw < _N, idraw, idraw - _N)
            w_syn = jnp.where((idraw < _N) & is_self,
                              jnp.full((_L,), 1.0, jnp.float32), _zeros16())
            rows16 = jnp.where(is_real, r16, id_eff)
            cols16 = jnp.where(is_real, c16, id_eff)
            weff16 = jnp.where(is_real, wr, w_syn)
            rstage[sl] = rows16
            cstage[sl] = cols16
            wstage[sl] = weff16
            qsl = pl.ds(q * _L, _L)
            ridx[qsl] = rows16
            weff[qsl] = weff16
        pltpu.sync_copy(weff, deg_sh.at[ridx], add=True)
        return carry

    lax.fori_loop(0, _ACPT, _chunk, 0)

    esl = pl.ds(wid * _EPT, _EPT)
    o1 = pltpu.async_copy(rstage, rows_out.at[esl], sem)
    o2 = pltpu.async_copy(cstage, cols_out.at[esl], sem)
    o3 = pltpu.async_copy(wstage, weff_out.at[esl], sem)

    plsc.subcore_barrier()
    pltpu.sync_copy(deg_sh.at[nsl], deg_out.at[pl.ds(cid * _NP + sid * _NTILE, _NTILE)])
    o1.wait()
    o2.wait()
    o3.wait()


@functools.partial(
    pl.kernel,
    out_type=jax.ShapeDtypeStruct((_NC * _NP, _D), jnp.float32),
    mesh=_mesh,
    scratch_types=[
        pltpu.VMEM_SHARED((_NP, _D), jnp.float32),
        pltpu.VMEM_SHARED((_NP,), jnp.float32),
        pltpu.VMEM((_NP,), jnp.float32),
        pltpu.VMEM((_SCE,), jnp.int32),
        pltpu.VMEM((_SCE,), jnp.int32),
        pltpu.VMEM((_SCE,), jnp.float32),
        pltpu.VMEM((2, _SCR, _EC), jnp.int32),
        pltpu.VMEM((_EC,), jnp.int32),
        pltpu.VMEM((_EC, _D), jnp.float32),
        pltpu.VMEM((_EC, _D), jnp.float32),
        pltpu.VMEM((_NTILE,), jnp.float32),
        pltpu.VMEM((_NTILE,), jnp.float32),
        pltpu.SemaphoreType.DMA,
        pltpu.SemaphoreType.DMA,
        pltpu.SemaphoreType.DMA,
        pltpu.SemaphoreType.DMA,
    ],
    compiler_params=_params,
)
def _spmm_kernel(deg_hbm, row_hbm, col_hbm, w_hbm, x_hbm, out_hbm,
                 out_sh, dis_sh, dis_v, rstage, cstage, wstage,
                 ridx3, pidx, buf0, buf1, dbuf0, dbuf1,
                 semg0, semg1, sems0, sems1):
    cid = lax.axis_index("c")
    sid = lax.axis_index("s")
    wid = cid * _NS + sid

    izero = jnp.zeros((_L,), jnp.int32)

    def _zero(j, carry):
        for q in range(_D // _L):
            sl = pl.ds(q * _L, _L)
            buf0[j, sl] = _zeros16()
            buf1[j, sl] = _zeros16()
        return carry

    lax.fori_loop(0, _EC, _zero, 0)
    for q in range(_EC // _L):
        pidx[pl.ds(q * _L, _L)] = izero

    for k in range(6):
        pltpu.sync_copy(buf0, out_sh.at[pl.ds(sid * _NTILE + k * _EC, _EC)])
    pltpu.sync_copy(buf0.at[pl.ds(0, _NTILE - 6 * _EC)],
                    out_sh.at[pl.ds(sid * _NTILE + 6 * _EC, _NTILE - 6 * _EC)])

    nsl = pl.ds(sid * _NTILE, _NTILE)
    pltpu.sync_copy(deg_hbm.at[pl.ds(sid * _NTILE, _NTILE)], dbuf0)
    pltpu.sync_copy(deg_hbm.at[pl.ds(_NP + sid * _NTILE, _NTILE)], dbuf1)

    def _dis(i, carry):
        sl = pl.ds(i * _L, _L)
        d = dbuf0[sl] + dbuf1[sl]
        d = jnp.where(d > 0.0, d, d + 1.0)
        dbuf0[sl] = _rsqrt_vec(d)
        return carry

    lax.fori_loop(0, _NTILE // _L, _dis, 0)
    pltpu.sync_copy(dbuf0, dis_sh.at[nsl])
    plsc.subcore_barrier()
    pltpu.sync_copy(dis_sh, dis_v)

    pltpu.async_copy(buf1, out_sh.at[pidx], sems1, add=True)

    def _wait(sem, dst):
        pltpu.make_async_copy(x_hbm.at[pl.ds(0, _EC)], dst, sem).wait()

    def _scale(buf, j):
        def _g(g, carry):
            sl = pl.ds(j * _EC + g * _L, _L)
            r16 = rstage[sl]
            c16 = cstage[sl]
            w16 = wstage[sl]
            wn16 = plsc.load_gather(dis_v, [r16]) * w16 * plsc.load_gather(dis_v, [c16])
            for e16 in range(_L):
                s = wn16[e16]
                e = g * _L + e16
                for q in range(_D // _L):
                    qsl = pl.ds(q * _L, _L)
                    buf[e, qsl] = buf[e, qsl] * s
            return carry

        lax.fori_loop(0, _EC // _L, _g, 0)

    base = wid * _EPT
    for sc in range(_CPT // _SCR):
        par = sc % 2
        sbase = base + sc * _SCE
        pltpu.sync_copy(row_hbm.at[pl.ds(sbase, _SCE)], rstage)
        pltpu.sync_copy(col_hbm.at[pl.ds(sbase, _SCE)], cstage)
        pltpu.sync_copy(w_hbm.at[pl.ds(sbase, _SCE)], wstage)

        def _bld(j, carry):
            for q in range(_EC // _L):
                ridx3[par, j, pl.ds(q * _L, _L)] = rstage[pl.ds(j * _EC + q * _L, _L)]
            return carry

        lax.fori_loop(0, _SCR, _bld, 0)

        pltpu.async_copy(x_hbm.at[cstage.at[pl.ds(0, _EC)]], buf0, semg0)

        def _pair(p, carry):
            a = 2 * p
            b = a + 1
            _wait(sems1, buf1)
            pltpu.async_copy(
                x_hbm.at[cstage.at[pl.ds(b * _EC, _EC)]], buf1, semg1)
            _wait(semg0, buf0)
            _scale(buf0, a)
            pltpu.async_copy(buf0, out_sh.at[ridx3.at[par, a]], sems0, add=True)
            _wait(semg1, buf1)
            _wait(sems0, buf0)

            @pl.when(p < _NPAIR - 1)
            def _prefetch():
                pltpu.async_copy(
                    x_hbm.at[cstage.at[pl.ds((a + 2) * _EC, _EC)]], buf0, semg0)

            _scale(buf1, b)
            pltpu.async_copy(buf1, out_sh.at[ridx3.at[par, b]], sems1, add=True)
            return carry

        lax.fori_loop(0, _NPAIR, _pair, 0)

    _wait(sems1, buf1)
    plsc.subcore_barrier()
    pltpu.sync_copy(out_sh.at[nsl], out_hbm.at[pl.ds(cid * _NP + sid * _NTILE, _NTILE)])


def _add_body(a_ref, b_ref, o_ref):
    o_ref[...] = a_ref[...] + b_ref[...]


_BR = 1000


def _combine(p0, p1):
    return pl.pallas_call(
        _add_body,
        out_shape=jax.ShapeDtypeStruct((_N, _D), jnp.float32),
        grid=(_N // _BR,),
        in_specs=[pl.BlockSpec((_BR, _D), lambda i: (i, 0))] * 2,
        out_specs=pl.BlockSpec((_BR, _D), lambda i: (i, 0)),
    )(p0, p1)


def kernel(x, edge_index, edge_attr):
    assert x.shape == (_N, _D)
    ei = edge_index.astype(jnp.int32).reshape(2 * _E)
    w = edge_attr.astype(jnp.float32)

    deg, rows, cols, weff = _deg_kernel(ei, w)
    parts = _spmm_kernel(deg, rows, cols, weff, x)
    return _combine(parts[:_N], parts[_NP:_NP + _N])
